# Initial kernel scaffold; baseline (speedup 1.0000x reference)
import jax
import jax.numpy as jnp
from jax.experimental import pallas as pl


def kernel(x, position_embeddings):
    raise NotImplementedError("write your pallas kernel here")


# speedup vs baseline: 1.2312x; 1.2312x over previous
import jax
import jax.numpy as jnp
from jax.experimental import pallas as pl

_BS = 256


def _body(x_ref, pe_ref, o_ref):
    pe = pe_ref[...]
    o_ref[...] = x_ref[...] + jnp.concatenate([pe, pe, pe, pe], axis=1)


def kernel(x, position_embeddings):
    S, B, D = x.shape
    x2 = x.reshape(S, B * D)
    out = pl.pallas_call(
        _body,
        grid=(S // _BS,),
        in_specs=[
            pl.BlockSpec((_BS, B * D), lambda i: (i, 0)),
            pl.BlockSpec((_BS, D), lambda i: (i, 0)),
        ],
        out_specs=pl.BlockSpec((_BS, B * D), lambda i: (i, 0)),
        out_shape=jax.ShapeDtypeStruct((S, B * D), x.dtype),
    )(x2, position_embeddings[:S])
    return out.reshape(S, B, D)
